# baseline (speedup 1.0000x reference)
import functools
import jax
import jax.numpy as jnp
from jax import lax
from jax.experimental import pallas as pl
from jax.experimental.pallas import tpu as pltpu, tpu_sc as plsc

VOCAB = 100000
DIM = 1024
NPOS = 2048
BATCH = 4

NC = 2
NS = 16
NW = NC * NS
LANES = 16

POS_PER_W = NPOS // NW
CP = 32
NCHUNK = POS_PER_W // CP
DVEC = DIM // LANES


def _body(tok_hbm, w_hbm, pos_hbm, out_hbm, idx_v, pos_v, rows_v, sem):
    wid = lax.axis_index("s") * NC + lax.axis_index("c")
    p_base = wid * POS_PER_W

    for b in range(BATCH):
        pltpu.sync_copy(
            tok_hbm.at[pl.ds(b * NPOS + p_base, POS_PER_W)],
            idx_v.at[pl.ds(b * POS_PER_W, POS_PER_W)],
        )

    for c in range(NCHUNK):
        p0 = p_base + c * CP
        pltpu.sync_copy(pos_hbm.at[pl.ds(p0, CP)], pos_v)
        for b in range(BATCH):
            pltpu.async_copy(
                w_hbm.at[idx_v.at[pl.ds(b * POS_PER_W + c * CP, CP)]],
                rows_v, sem
            ).wait()

            @pl.loop(0, CP)
            def _row(r):
                @pl.loop(0, DVEC, unroll=8)
                def _vec(d):
                    off = d * LANES
                    rows_v[r, pl.ds(off, LANES)] = (
                        rows_v[r, pl.ds(off, LANES)]
                        + pos_v[r, pl.ds(off, LANES)]
                    )

            pltpu.sync_copy(
                rows_v, out_hbm.at[pl.ds(b * NPOS + p0, CP)]
            )


@jax.jit
def _embed(tokens, W, pos_enc):
    mesh = plsc.VectorSubcoreMesh(
        core_axis_name="c", subcore_axis_name="s",
        num_cores=NC, num_subcores=NS,
    )
    run = pl.kernel(
        _body,
        out_type=jax.ShapeDtypeStruct((BATCH * NPOS, DIM), jnp.float32),
        mesh=mesh,
        scratch_types=[
            pltpu.VMEM((BATCH * POS_PER_W,), jnp.int32),
            pltpu.VMEM((CP, DIM), jnp.float32),
            pltpu.VMEM((CP, DIM), jnp.float32),
            pltpu.SemaphoreType.DMA,
        ],
    )
    out = run(tokens.reshape(-1), W, pos_enc)
    return out.reshape(BATCH, NPOS, DIM)


def kernel(tokens, W, pos_enc):
    return _embed(tokens.astype(jnp.int32), W, pos_enc)


# speedup vs baseline: 2.1484x; 2.1484x over previous
import jax
import jax.numpy as jnp
from jax import lax
from jax.experimental import pallas as pl
from jax.experimental.pallas import tpu as pltpu, tpu_sc as plsc

VOCAB = 100000
DIM = 1024
NPOS = 2048
BATCH = 4

NC = 2
NS = 16
NW = NC * NS
LANES = 16

POS_PER_W = NPOS // NW
CP = 32
NCHUNK = POS_PER_W // CP
NSTEP = NCHUNK * BATCH
DVEC = DIM // LANES


def _body(tok_hbm, w_hbm, pos_hbm, out_hbm,
          idx_v, pos_v, rows0, rows1, g0, g1, s0, s1):
    wid = lax.axis_index("s") * NC + lax.axis_index("c")
    p_base = wid * POS_PER_W
    rows = (rows0, rows1)
    gsem = (g0, g1)
    ssem = (s0, s1)

    for b in range(BATCH):
        pltpu.sync_copy(
            tok_hbm.at[pl.ds(b * NPOS + p_base, POS_PER_W)],
            idx_v.at[pl.ds(b * POS_PER_W, POS_PER_W)],
        )

    def gather(k, buf):
        c, b = divmod(k, BATCH)
        return pltpu.async_copy(
            w_hbm.at[idx_v.at[pl.ds(b * POS_PER_W + c * CP, CP)]],
            rows[buf], gsem[buf],
        )

    stores = [None, None]
    g = gather(0, 0)
    for k in range(NSTEP):
        cur = k % 2
        nxt = 1 - cur
        c, b = divmod(k, BATCH)
        if b == 0:
            pltpu.sync_copy(pos_hbm.at[pl.ds(p_base + c * CP, CP)], pos_v)
        if k + 1 < NSTEP:
            if stores[nxt] is not None:
                stores[nxt].wait()
                stores[nxt] = None
            g_next = gather(k + 1, nxt)
        g.wait()

        @pl.loop(0, CP)
        def _row(r):
            for d in range(DVEC):
                off = d * LANES
                plsc.addupdate(
                    rows[cur].at[r, pl.ds(off, LANES)],
                    pos_v[r, pl.ds(off, LANES)],
                )

        stores[cur] = pltpu.async_copy(
            rows[cur],
            out_hbm.at[pl.ds(b * NPOS + p_base + c * CP, CP)],
            ssem[cur],
        )
        if k + 1 < NSTEP:
            g = g_next
    for st in stores:
        if st is not None:
            st.wait()


@jax.jit
def _embed(tokens, W, pos_enc):
    mesh = plsc.VectorSubcoreMesh(
        core_axis_name="c", subcore_axis_name="s",
        num_cores=NC, num_subcores=NS,
    )
    run = pl.kernel(
        _body,
        out_type=jax.ShapeDtypeStruct((BATCH * NPOS, DIM), jnp.float32),
        mesh=mesh,
        scratch_types=[
            pltpu.VMEM((BATCH * POS_PER_W,), jnp.int32),
            pltpu.VMEM((CP, DIM), jnp.float32),
            pltpu.VMEM((CP, DIM), jnp.float32),
            pltpu.VMEM((CP, DIM), jnp.float32),
            pltpu.SemaphoreType.DMA,
            pltpu.SemaphoreType.DMA,
            pltpu.SemaphoreType.DMA,
            pltpu.SemaphoreType.DMA,
        ],
    )
    out = run(tokens.reshape(-1), W, pos_enc)
    return out.reshape(BATCH, NPOS, DIM)


def kernel(tokens, W, pos_enc):
    return _embed(tokens.astype(jnp.int32), W, pos_enc)
